# baseline (speedup 1.0000x reference)
import functools

import jax
import jax.numpy as jnp
from jax.experimental import pallas as pl
from jax.experimental.pallas import tpu as pltpu

MAXD = 5


def _body(nn_ref, dist_ref, tc_ref, tb_ref, out_ref, *, rows, n, h):
    b = pl.program_id(0)
    r = pl.program_id(1)
    nn = nn_ref[b]
    d = dist_ref[0]
    idx = jnp.clip(d, -1, MAXD) + 1
    jio = jax.lax.broadcasted_iota(jnp.int32, (rows, n), 1)
    iio = jax.lax.broadcasted_iota(jnp.int32, (rows, n), 0) + r * rows
    valid = (jio < nn) & (iio < nn)
    idx = jnp.where(valid, idx, 0)

    tsrc = jnp.broadcast_to(tc_ref[0], (rows, h, 128))
    for c in range(n // 128):
        sl = slice(c * 128, (c + 1) * 128)
        idx8 = jnp.broadcast_to(idx[:, None, sl], (rows, h, 128))
        if c == 0:
            val = jnp.take_along_axis(tsrc, idx8, axis=2)
        else:
            val = jnp.zeros((rows, h, 128), jnp.float32)
            for k in range(1, MAXD + 2):
                val = jnp.where(idx8 == k, tb_ref[k], val)
        out_ref[0, :, :, sl] = val


def kernel(dist, batch_num_nodes, embedding_table):
    B, N, _ = dist.shape
    K, H = embedding_table.shape
    tz = embedding_table.at[0].set(0.0)
    tc = jnp.zeros((1, H, 128), jnp.float32).at[0, :, :K].set(tz.T)
    tb = jnp.broadcast_to(embedding_table[:, :, None], (K, H, 128))
    ROWS = 512
    grid = (B, N // ROWS)

    out = pl.pallas_call(
        functools.partial(_body, rows=ROWS, n=N, h=H),
        grid_spec=pltpu.PrefetchScalarGridSpec(
            num_scalar_prefetch=1,
            grid=grid,
            in_specs=[
                pl.BlockSpec((1, ROWS, N), lambda b, r, nn: (b, r, 0)),
                pl.BlockSpec((1, H, 128), lambda b, r, nn: (0, 0, 0)),
                pl.BlockSpec((K, H, 128), lambda b, r, nn: (0, 0, 0)),
            ],
            out_specs=pl.BlockSpec(
                (1, ROWS, H, N), lambda b, r, nn: (b, r, 0, 0)
            ),
        ),
        out_shape=jax.ShapeDtypeStruct((B, N, H, N), jnp.float32),
        compiler_params=pltpu.CompilerParams(
            dimension_semantics=("parallel", "parallel")
        ),
    )(batch_num_nodes.astype(jnp.int32), dist, tc, tb)
    return jnp.transpose(out, (0, 1, 3, 2))


# speedup vs baseline: 1.2701x; 1.2701x over previous
import functools

import jax
import jax.numpy as jnp
from jax.experimental import pallas as pl
from jax.experimental.pallas import tpu as pltpu

MAXD = 5


def _body(nn_ref, dist_ref, tc_ref, tb_ref, out_ref, *, rows, n, h):
    b = pl.program_id(0)
    r = pl.program_id(1)
    nn = nn_ref[b]
    d = dist_ref[0]
    idx = jnp.clip(d, -1, MAXD) + 1
    jio = jax.lax.broadcasted_iota(jnp.int32, (rows, n), 1)
    iio = jax.lax.broadcasted_iota(jnp.int32, (rows, n), 0) + r * rows
    valid = (jio < nn) & (iio < nn)
    idx = jnp.where(valid, idx, 0)

    k7 = MAXD + 2
    tsrc = jnp.broadcast_to(tc_ref[0], (rows, h, 128))
    idx8 = jnp.broadcast_to(idx[:, None, :128], (rows, h, 128))
    out_ref[0, :, :, :128] = jnp.take_along_axis(tsrc, idx8, axis=2)
    w = n - 128
    lhs = jnp.broadcast_to(tb_ref[0][None], (rows, h, k7))
    kio = jax.lax.broadcasted_iota(jnp.int32, (rows, k7, w), 1)
    oh = (idx[:, None, 128:] == kio).astype(jnp.bfloat16)
    out_ref[0, :, :, 128:] = jax.lax.dot_general(
        lhs,
        oh,
        (((2,), (1,)), ((0,), (0,))),
        preferred_element_type=jnp.float32,
    )


def kernel(dist, batch_num_nodes, embedding_table):
    B, N, _ = dist.shape
    K, H = embedding_table.shape
    tz = embedding_table.at[0].set(0.0)
    tc = jnp.zeros((1, H, 128), jnp.float32).at[0, :, :K].set(tz.T)
    tb = tz.T.astype(jnp.bfloat16)[None]
    ROWS = 512
    grid = (B, N // ROWS)

    out = pl.pallas_call(
        functools.partial(_body, rows=ROWS, n=N, h=H),
        grid_spec=pltpu.PrefetchScalarGridSpec(
            num_scalar_prefetch=1,
            grid=grid,
            in_specs=[
                pl.BlockSpec((1, ROWS, N), lambda b, r, nn: (b, r, 0)),
                pl.BlockSpec((1, H, 128), lambda b, r, nn: (0, 0, 0)),
                pl.BlockSpec((1, H, K), lambda b, r, nn: (0, 0, 0)),
            ],
            out_specs=pl.BlockSpec(
                (1, ROWS, H, N), lambda b, r, nn: (b, r, 0, 0)
            ),
        ),
        out_shape=jax.ShapeDtypeStruct((B, N, H, N), jnp.float32),
        compiler_params=pltpu.CompilerParams(
            dimension_semantics=("parallel", "parallel")
        ),
    )(batch_num_nodes.astype(jnp.int32), dist, tc, tb)
    return jnp.transpose(out, (0, 1, 3, 2))
